# baseline (speedup 1.0000x reference)
import jax
import jax.numpy as jnp
from jax.experimental import pallas as pl
from jax.experimental.pallas import tpu as pltpu

_KS = (1, 5, 10, 50, 100)
_NQ = 1024
_D = 128
_BN = 4096
_NCHUNK = _BN // 128


def _chunk_tree_sum(m):
    chunks = [m[:, j * 128:(j + 1) * 128] for j in range(m.shape[1] // 128)]
    while len(chunks) > 1:
        chunks = [chunks[j] + chunks[j + 1] for j in range(0, len(chunks), 2)]
    return chunks[0]


def _fused_kernel(q_ref, t_ref, c_ref, accs_ref, acc_ref, pos_ref,
                  *, n_valid, n_blocks):
    i = pl.program_id(0)

    @pl.when(i == 0)
    def _init():
        pos_ref[...] = jnp.sum(q_ref[...] * t_ref[...], axis=1, keepdims=True)
        acc_ref[...] = jnp.zeros_like(acc_ref)

    pos = pos_ref[...]
    scores = jax.lax.dot_general(
        q_ref[...], c_ref[...], (((1,), (1,)), ((), ())),
        preferred_element_type=jnp.float32)
    mask = scores > pos
    acc_ref[...] += jnp.sum(mask, axis=1, keepdims=True)

    @pl.when(i == n_blocks - 1)
    def _final():
        rem = n_valid - (n_blocks - 1) * _BN
        col = jax.lax.broadcasted_iota(jnp.int32, mask.shape, 1)
        excess = jnp.logical_and(mask, col >= rem)
        cnt = acc_ref[...] - jnp.sum(excess, axis=1, keepdims=True)
        lane = jax.lax.broadcasted_iota(jnp.int32, (_NQ, 128), 1)
        thr = jnp.full((_NQ, 128), jnp.iinfo(jnp.int32).max, dtype=jnp.int32)
        for j, kv in enumerate(_KS):
            thr = jnp.where(lane == j, kv, thr)
        ind = (cnt < thr).astype(jnp.float32)
        accs_ref[...] = jnp.sum(ind, axis=0, keepdims=True) * (1.0 / _NQ)


def kernel(query_embeddings, true_candidate_embeddings, candidates):
    n = candidates.shape[0]
    n_blocks = pl.cdiv(n, _BN)
    accs = pl.pallas_call(
        lambda qr, tr, cr, ar, sr, pr: _fused_kernel(
            qr, tr, cr, ar, sr, pr, n_valid=n, n_blocks=n_blocks),
        grid=(n_blocks,),
        in_specs=[
            pl.BlockSpec((_NQ, _D), lambda i: (0, 0)),
            pl.BlockSpec((_NQ, _D), lambda i: (0, 0)),
            pl.BlockSpec((_BN, _D), lambda i: (i, 0)),
        ],
        out_specs=pl.BlockSpec((1, 128), lambda i: (0, 0)),
        out_shape=jax.ShapeDtypeStruct((1, 128), jnp.float32),
        scratch_shapes=[
            pltpu.VMEM((_NQ, 1), jnp.int32),
            pltpu.VMEM((_NQ, 1), jnp.float32),
        ],
    )(query_embeddings, true_candidate_embeddings, candidates)
    return accs[0, : len(_KS)]


# speedup vs baseline: 1.0565x; 1.0565x over previous
import jax
import jax.numpy as jnp
from jax.experimental import pallas as pl
from jax.experimental.pallas import tpu as pltpu

_KS = (1, 5, 10, 50, 100)
_NQ = 1024
_D = 128
_BN = 4096
_NCHUNK = _BN // 128


def _chunk_tree_sum(m):
    chunks = [m[:, j * 128:(j + 1) * 128] for j in range(m.shape[1] // 128)]
    while len(chunks) > 1:
        chunks = [chunks[j] + chunks[j + 1] for j in range(0, len(chunks), 2)]
    return chunks[0]


def _fused_kernel(q_ref, t_ref, c_ref, accs_ref, acc_ref, pos_ref,
                  *, n_valid, n_blocks):
    i = pl.program_id(0)

    @pl.when(i == 0)
    def _init():
        pos_ref[...] = jnp.sum(q_ref[...] * t_ref[...], axis=1, keepdims=True)
        acc_ref[...] = jnp.zeros_like(acc_ref)

    pos = pos_ref[...]
    scores = jax.lax.dot_general(
        q_ref[...], c_ref[...], (((1,), (1,)), ((), ())),
        preferred_element_type=jnp.float32)
    mask = scores > pos
    m = jnp.where(mask, 1.0, 0.0)
    acc_ref[...] += _chunk_tree_sum(m)

    @pl.when(i == n_blocks - 1)
    def _final():
        rem = n_valid - (n_blocks - 1) * _BN
        col = jax.lax.broadcasted_iota(jnp.int32, mask.shape, 1)
        excess = jnp.where(jnp.logical_and(mask, col >= rem), 1.0, 0.0)
        cnt = (jnp.sum(acc_ref[...], axis=1, keepdims=True)
               - jnp.sum(excess, axis=1, keepdims=True))
        lane = jax.lax.broadcasted_iota(jnp.int32, (_NQ, 128), 1)
        thr = jnp.full((_NQ, 128), jnp.inf, dtype=jnp.float32)
        for j, kv in enumerate(_KS):
            thr = jnp.where(lane == j, float(kv), thr)
        ind = (cnt < thr).astype(jnp.float32)
        accs_ref[...] = jnp.sum(ind, axis=0, keepdims=True) * (1.0 / _NQ)


def kernel(query_embeddings, true_candidate_embeddings, candidates):
    n = candidates.shape[0]
    n_blocks = pl.cdiv(n, _BN)
    accs = pl.pallas_call(
        lambda qr, tr, cr, ar, sr, pr: _fused_kernel(
            qr, tr, cr, ar, sr, pr, n_valid=n, n_blocks=n_blocks),
        grid=(n_blocks,),
        in_specs=[
            pl.BlockSpec((_NQ, _D), lambda i: (0, 0)),
            pl.BlockSpec((_NQ, _D), lambda i: (0, 0)),
            pl.BlockSpec((_BN, _D), lambda i: (i, 0)),
        ],
        out_specs=pl.BlockSpec((1, 128), lambda i: (0, 0)),
        out_shape=jax.ShapeDtypeStruct((1, 128), jnp.float32),
        scratch_shapes=[
            pltpu.VMEM((_NQ, 128), jnp.float32),
            pltpu.VMEM((_NQ, 1), jnp.float32),
        ],
    )(query_embeddings, true_candidate_embeddings, candidates)
    return accs[0, : len(_KS)]
